# baseline (speedup 1.0000x reference)
import functools

import jax
import jax.numpy as jnp
from jax import lax
from jax.experimental import pallas as pl
from jax.experimental.pallas import tpu as pltpu
from jax.experimental.pallas import tpu_sc as plsc


_NSPLIT = 4


def _mm_body(idx_ref, x_ref, *refs):
    w_refs = refs[:_NSPLIT]
    bias_ref, o_ref = refs[_NSPLIT], refs[_NSPLIT + 1]
    e = pl.program_id(0)
    chunk = w_refs[0].shape[1]
    mask = idx_ref[...] == e
    xm = jnp.where(mask, x_ref[...], 0.0)
    for j, wr in enumerate(w_refs):
        contrib = lax.dot_general(
            xm, wr[0],
            dimension_numbers=(((1,), (1,)), ((), ())),
            preferred_element_type=jnp.float32,
        )
        sl = pl.ds(j * chunk, chunk)

        @pl.when(e == 0)
        def _(contrib=contrib, sl=sl):
            o_ref[:, sl] = bias_ref[:, sl] + contrib

        @pl.when(e != 0)
        def _(contrib=contrib, sl=sl):
            o_ref[:, sl] += contrib


def _switch_matmul(idx2d, x, W, bias_g):
    B, IN = x.shape
    E, OUT, _ = W.shape
    chunk = OUT // _NSPLIT
    w_specs = [
        pl.BlockSpec((1, chunk, IN), lambda e, j=j: (e, j, 0))
        for j in range(_NSPLIT)
    ]
    return pl.pallas_call(
        _mm_body,
        grid=(E,),
        in_specs=[
            pl.BlockSpec((B, 1), lambda e: (0, 0)),
            pl.BlockSpec((B, IN), lambda e: (0, 0)),
            *w_specs,
            pl.BlockSpec((B, OUT), lambda e: (0, 0)),
        ],
        out_specs=pl.BlockSpec((B, OUT), lambda e: (0, 0)),
        out_shape=jax.ShapeDtypeStruct((B, OUT), jnp.float32),
    )(idx2d, x, *([W] * _NSPLIT), bias_g)


def _bias_gather(b, idx):
    E, OUT = b.shape
    B = idx.shape[0]
    n_workers = 16
    per_w = B // n_workers
    info = plsc.get_sparse_core_info()
    nc = info.num_cores
    mesh = plsc.VectorSubcoreMesh(core_axis_name="c", subcore_axis_name="s")

    @functools.partial(
        pl.kernel,
        mesh=mesh,
        out_type=jax.ShapeDtypeStruct((B, OUT), jnp.float32),
        scratch_types=[
            pltpu.VMEM((per_w,), jnp.int32),
            pltpu.VMEM((per_w, OUT), jnp.float32),
            pltpu.SemaphoreType.DMA,
        ],
    )
    def k(b_hbm, idx_hbm, out_hbm, idx_v, rows_v, sem):
        wid = lax.axis_index("s") * nc + lax.axis_index("c")

        @pl.when(wid < n_workers)
        def _():
            base = wid * per_w
            pltpu.sync_copy(idx_hbm.at[pl.ds(base, per_w)], idx_v)
            pltpu.async_copy(b_hbm.at[idx_v], rows_v, sem).wait()
            pltpu.sync_copy(rows_v, out_hbm.at[pl.ds(base, per_w)])

    return k(b, idx)


def kernel(x, indices, W, b):
    idx = indices.astype(jnp.int32)
    bias_g = _bias_gather(b, idx)
    idx2d = idx.reshape(-1, 1)
    return _switch_matmul(idx2d, x, W, bias_g)


# speedup vs baseline: 1.3605x; 1.3605x over previous
import functools

import jax
import jax.numpy as jnp
from jax import lax
from jax.experimental import pallas as pl
from jax.experimental.pallas import tpu as pltpu
from jax.experimental.pallas import tpu_sc as plsc


_NBUF = 4


def _mm_body(idx_ref, x_ref, w_hbm, bias_ref, o_ref, wbuf, sems):
    e = pl.program_id(0)
    n = pl.num_programs(0)

    @pl.when(e == 0)
    def _():
        for k in range(_NBUF):
            pltpu.make_async_copy(w_hbm.at[k], wbuf.at[k], sems.at[k]).start()

    slot = lax.rem(e, _NBUF)
    pltpu.make_async_copy(w_hbm.at[e], wbuf.at[slot], sems.at[slot]).wait()

    mask = idx_ref[...] == e
    xm = jnp.where(mask, x_ref[...], 0.0)
    contrib = lax.dot_general(
        xm, wbuf[slot],
        dimension_numbers=(((1,), (1,)), ((), ())),
        preferred_element_type=jnp.float32,
    )

    @pl.when(e == 0)
    def _():
        o_ref[...] = bias_ref[...] + contrib

    @pl.when(e != 0)
    def _():
        o_ref[...] += contrib

    nxt = e + _NBUF

    @pl.when(nxt < n)
    def _():
        pltpu.make_async_copy(w_hbm.at[nxt], wbuf.at[slot], sems.at[slot]).start()


def _switch_matmul(idx2d, x, W, bias_g):
    B, IN = x.shape
    E, OUT, _ = W.shape
    return pl.pallas_call(
        _mm_body,
        grid=(E,),
        in_specs=[
            pl.BlockSpec((B, 1), lambda e: (0, 0)),
            pl.BlockSpec((B, IN), lambda e: (0, 0)),
            pl.BlockSpec(memory_space=pl.ANY),
            pl.BlockSpec((B, OUT), lambda e: (0, 0)),
        ],
        out_specs=pl.BlockSpec((B, OUT), lambda e: (0, 0)),
        out_shape=jax.ShapeDtypeStruct((B, OUT), jnp.float32),
        scratch_shapes=[
            pltpu.VMEM((_NBUF, OUT, IN), jnp.float32),
            pltpu.SemaphoreType.DMA((_NBUF,)),
        ],
    )(idx2d, x, W, bias_g)


def _bias_gather(b, idx):
    E, OUT = b.shape
    B = idx.shape[0]
    n_workers = 16
    per_w = B // n_workers
    info = plsc.get_sparse_core_info()
    nc = info.num_cores
    mesh = plsc.VectorSubcoreMesh(core_axis_name="c", subcore_axis_name="s")

    @functools.partial(
        pl.kernel,
        mesh=mesh,
        out_type=jax.ShapeDtypeStruct((B, OUT), jnp.float32),
        scratch_types=[
            pltpu.VMEM((per_w,), jnp.int32),
            pltpu.VMEM((per_w, OUT), jnp.float32),
            pltpu.SemaphoreType.DMA,
        ],
    )
    def k(b_hbm, idx_hbm, out_hbm, idx_v, rows_v, sem):
        wid = lax.axis_index("s") * nc + lax.axis_index("c")

        @pl.when(wid < n_workers)
        def _():
            base = wid * per_w
            pltpu.sync_copy(idx_hbm.at[pl.ds(base, per_w)], idx_v)
            pltpu.async_copy(b_hbm.at[idx_v], rows_v, sem).wait()
            pltpu.sync_copy(rows_v, out_hbm.at[pl.ds(base, per_w)])

    return k(b, idx)


def kernel(x, indices, W, b):
    idx = indices.astype(jnp.int32)
    bias_g = _bias_gather(b, idx)
    idx2d = idx.reshape(-1, 1)
    return _switch_matmul(idx2d, x, W, bias_g)
